# baseline (speedup 1.0000x reference)
import functools

import jax
import jax.numpy as jnp
from jax import lax
from jax.experimental import pallas as pl
from jax.experimental.pallas import tpu as pltpu
from jax.experimental.pallas import tpu_sc as plsc

B = 16384
H = 50
D = 64
O = 64
NC = 2
NS = 16
NW = NC * NS
RPW = B // NW
GL = 56
IB = 8
XB = 64
NSLICE = D // 16


def _pooled_sc(x, emb):
    mesh = plsc.VectorSubcoreMesh(core_axis_name="c", subcore_axis_name="s")

    @functools.partial(
        pl.kernel,
        mesh=mesh,
        out_type=jax.ShapeDtypeStruct((B, D), jnp.float32),
        scratch_types=[
            pltpu.SMEM((IB, H), jnp.int32),
            pltpu.VMEM((XB, H), jnp.int32),
            pltpu.VMEM_SHARED((NS, XB, H), jnp.int32),
            pltpu.VMEM((GL, D), jnp.float32),
            pltpu.VMEM((IB, D), jnp.float32),
            pltpu.SemaphoreType.DMA,
        ],
    )
    def k(x_hbm, emb_hbm, out_hbm, idx_s, idx_v, shr_i, buf_v, out_b, sem):
        sid = lax.axis_index("s")
        wid = sid * NC + lax.axis_index("c")
        base = wid * RPW

        def blk_body(blk, carry):
            @pl.when(lax.rem(blk, XB // IB) == 0)
            def _():
                pltpu.sync_copy(
                    x_hbm.at[pl.ds(base + blk * IB, XB)], idx_v
                )
                pltpu.sync_copy(idx_v, shr_i.at[sid])

            pltpu.sync_copy(
                shr_i.at[sid, pl.ds(lax.rem(blk, XB // IB) * IB, IB)], idx_s
            )
            for rr in range(IB):
                for l in range(GL):
                    ridx = idx_s[rr, l if l < H else l - H]
                    pltpu.async_copy(emb_hbm.at[ridx], buf_v.at[l], sem)
                pltpu.make_async_copy(
                    emb_hbm.at[pl.ds(0, GL)], buf_v, sem
                ).wait()
                accs = [buf_v[0, pl.ds(j * 16, 16)] for j in range(NSLICE)]
                for l in range(1, H):
                    for j in range(NSLICE):
                        accs[j] = accs[j] + buf_v[l, pl.ds(j * 16, 16)]
                for j in range(NSLICE):
                    out_b[rr, pl.ds(j * 16, 16)] = accs[j] * (1.0 / H)
            pltpu.sync_copy(out_b, out_hbm.at[pl.ds(base + blk * IB, IB)])
            return carry

        lax.fori_loop(0, RPW // IB, blk_body, 0)

    return k(x, emb)


def _linear_tc(pooled, W, b):
    BM = 2048

    def mm(p_ref, w_ref, b_ref, o_ref):
        o_ref[...] = (
            jnp.dot(p_ref[...], w_ref[...], preferred_element_type=jnp.float32)
            + b_ref[...]
        )

    return pl.pallas_call(
        mm,
        grid=(B // BM,),
        in_specs=[
            pl.BlockSpec((BM, D), lambda i: (i, 0)),
            pl.BlockSpec((D, O), lambda i: (0, 0)),
            pl.BlockSpec((1, O), lambda i: (0, 0)),
        ],
        out_specs=pl.BlockSpec((BM, O), lambda i: (i, 0)),
        out_shape=jax.ShapeDtypeStruct((B, O), jnp.float32),
    )(pooled, W, b.reshape(1, O))


def kernel(x, emb, W, b):
    pooled = _pooled_sc(x.astype(jnp.int32), emb)
    return _linear_tc(pooled, W, b)


# speedup vs baseline: 3.2290x; 3.2290x over previous
import functools

import jax
import jax.numpy as jnp
from jax import lax
from jax.experimental import pallas as pl
from jax.experimental.pallas import tpu as pltpu
from jax.experimental.pallas import tpu_sc as plsc

B = 16384
H = 50
D = 64
O = 64
NC = 2
NS = 16
NW = NC * NS
RPW = B // NW
NBUF = 8
HP = 64
GL = 56
NSLICE = D // 16


def _pooled_sc(x, emb):
    mesh = plsc.VectorSubcoreMesh(core_axis_name="c", subcore_axis_name="s")

    @functools.partial(
        pl.kernel,
        mesh=mesh,
        out_type=jax.ShapeDtypeStruct((B, D), jnp.float32),
        compiler_params=pltpu.CompilerParams(use_tc_tiling_on_sc=False),
        scratch_types=[
            pltpu.VMEM((RPW, HP), jnp.int32),
            pltpu.VMEM((NBUF, GL, D), jnp.float32),
            pltpu.VMEM((RPW, D), jnp.float32),
        ] + [pltpu.SemaphoreType.DMA] * NBUF,
    )
    def k(x_hbm, emb_hbm, out_hbm, idx_v, buf_v, out_v, *sems):
        wid = lax.axis_index("s") * NC + lax.axis_index("c")
        pltpu.sync_copy(x_hbm.at[pl.ds(wid * RPW, RPW)], idx_v)

        def start(row, s):
            pltpu.async_copy(
                emb_hbm.at[idx_v.at[row, pl.ds(0, GL)]], buf_v.at[s], sems[s]
            )

        def wait(s):
            pltpu.make_async_copy(
                emb_hbm.at[idx_v.at[0, pl.ds(0, GL)]], buf_v.at[s], sems[s]
            ).wait()

        for s in range(NBUF):
            start(s, s)

        def body(i, carry):
            for s in range(NBUF):
                r = i * NBUF + s
                wait(s)
                accs = [buf_v[s, 0, pl.ds(j * 16, 16)] for j in range(NSLICE)]
                for l in range(1, H):
                    for j in range(NSLICE):
                        accs[j] = accs[j] + buf_v[s, l, pl.ds(j * 16, 16)]

                @pl.when(r + NBUF < RPW)
                def _():
                    start(r + NBUF, s)

                for j in range(NSLICE):
                    out_v[r, pl.ds(j * 16, 16)] = accs[j] * (1.0 / H)
            return carry

        lax.fori_loop(0, RPW // NBUF, body, 0)
        pltpu.sync_copy(out_v, out_hbm.at[pl.ds(wid * RPW, RPW)])

    return k(x, emb)


def _linear_tc(pooled, W, b):
    BM = 2048

    def mm(p_ref, w_ref, b_ref, o_ref):
        o_ref[...] = (
            jnp.dot(p_ref[...], w_ref[...], preferred_element_type=jnp.float32)
            + b_ref[...]
        )

    return pl.pallas_call(
        mm,
        grid=(B // BM,),
        in_specs=[
            pl.BlockSpec((BM, D), lambda i: (i, 0)),
            pl.BlockSpec((D, O), lambda i: (0, 0)),
            pl.BlockSpec((1, O), lambda i: (0, 0)),
        ],
        out_specs=pl.BlockSpec((BM, O), lambda i: (i, 0)),
        out_shape=jax.ShapeDtypeStruct((B, O), jnp.float32),
    )(pooled, W, b.reshape(1, O))


def kernel(x, emb, W, b):
    xp = jnp.pad(x.astype(jnp.int32), ((0, 0), (0, HP - H)), mode="wrap")
    pooled = _pooled_sc(xp, emb)
    return _linear_tc(pooled, W, b)


# speedup vs baseline: 3.2559x; 1.0083x over previous
import functools

import jax
import jax.numpy as jnp
from jax import lax
from jax.experimental import pallas as pl
from jax.experimental.pallas import tpu as pltpu
from jax.experimental.pallas import tpu_sc as plsc

B = 16384
H = 50
D = 64
O = 64
NC = 2
NS = 16
NW = NC * NS
RPW = B // NW
NBUF = 4
HP = 56
CB = 2
GL = CB * HP
NCHUNK = RPW // CB
NSLICE = D // 16


def _pooled_sc(x2, emb):
    mesh = plsc.VectorSubcoreMesh(core_axis_name="c", subcore_axis_name="s")

    @functools.partial(
        pl.kernel,
        mesh=mesh,
        out_type=jax.ShapeDtypeStruct((B, D), jnp.float32),
        compiler_params=pltpu.CompilerParams(use_tc_tiling_on_sc=False),
        scratch_types=[
            pltpu.VMEM((NCHUNK, GL), jnp.int32),
            pltpu.VMEM((NBUF, GL, D), jnp.float32),
            pltpu.VMEM((RPW, D), jnp.float32),
        ] + [pltpu.SemaphoreType.DMA] * NBUF,
    )
    def k(x_hbm, emb_hbm, out_hbm, idx_v, buf_v, out_v, *sems):
        wid = lax.axis_index("s") * NC + lax.axis_index("c")
        pltpu.sync_copy(x_hbm.at[pl.ds(wid * NCHUNK, NCHUNK)], idx_v)

        def start(chunk, s):
            pltpu.async_copy(emb_hbm.at[idx_v.at[chunk]], buf_v.at[s], sems[s])

        def wait(s):
            pltpu.make_async_copy(
                emb_hbm.at[idx_v.at[0]], buf_v.at[s], sems[s]
            ).wait()

        for s in range(NBUF):
            start(s, s)

        def body(i, carry):
            for s in range(NBUF):
                c = i * NBUF + s
                wait(s)
                for rr in range(CB):
                    accs = [buf_v[s, rr * HP, pl.ds(j * 16, 16)]
                            for j in range(NSLICE)]
                    for l in range(1, H):
                        for j in range(NSLICE):
                            accs[j] = accs[j] + buf_v[s, rr * HP + l,
                                                      pl.ds(j * 16, 16)]
                    for j in range(NSLICE):
                        out_v[c * CB + rr, pl.ds(j * 16, 16)] = (
                            accs[j] * (1.0 / H)
                        )

                @pl.when(c + NBUF < NCHUNK)
                def _():
                    start(c + NBUF, s)
            return carry

        lax.fori_loop(0, NCHUNK // NBUF, body, 0)
        pltpu.sync_copy(out_v, out_hbm.at[pl.ds(wid * RPW, RPW)])

    return k(x2, emb)


def _linear_tc(pooled, W, b):
    BM = 2048

    def mm(p_ref, w_ref, b_ref, o_ref):
        o_ref[...] = (
            jnp.dot(p_ref[...], w_ref[...], preferred_element_type=jnp.float32)
            + b_ref[...]
        )

    return pl.pallas_call(
        mm,
        grid=(B // BM,),
        in_specs=[
            pl.BlockSpec((BM, D), lambda i: (i, 0)),
            pl.BlockSpec((D, O), lambda i: (0, 0)),
            pl.BlockSpec((1, O), lambda i: (0, 0)),
        ],
        out_specs=pl.BlockSpec((BM, O), lambda i: (i, 0)),
        out_shape=jax.ShapeDtypeStruct((B, O), jnp.float32),
    )(pooled, W, b.reshape(1, O))


def kernel(x, emb, W, b):
    xp = jnp.pad(x.astype(jnp.int32), ((0, 0), (0, HP - H)), mode="wrap")
    pooled = _pooled_sc(xp.reshape(B // CB, GL), emb)
    return _linear_tc(pooled, W, b)


# speedup vs baseline: 3.4259x; 1.0522x over previous
import functools

import jax
import jax.numpy as jnp
from jax import lax
from jax.experimental import pallas as pl
from jax.experimental.pallas import tpu as pltpu
from jax.experimental.pallas import tpu_sc as plsc

B = 16384
H = 50
D = 64
O = 64
NC = 2
NS = 16
NW = NC * NS
RPW = B // NW
NBUF = 4
HP = 64
GL = 56
NSLICE = D // 16


def _pooled_sc(x, emb):
    mesh = plsc.VectorSubcoreMesh(core_axis_name="c", subcore_axis_name="s")

    @functools.partial(
        pl.kernel,
        mesh=mesh,
        out_type=jax.ShapeDtypeStruct((B, D), jnp.float32),
        compiler_params=pltpu.CompilerParams(use_tc_tiling_on_sc=False),
        scratch_types=[
            pltpu.VMEM((RPW, HP), jnp.int32),
            pltpu.VMEM((NBUF, GL, D), jnp.float32),
            pltpu.VMEM((RPW, D), jnp.float32),
            pltpu.SemaphoreType.DMA,
            pltpu.SemaphoreType.DMA,
            pltpu.SemaphoreType.DMA,
            pltpu.SemaphoreType.DMA,
        ],
    )
    def k(x_hbm, emb_hbm, out_hbm, idx_v, buf_v, out_v, *sems):
        wid = lax.axis_index("s") * NC + lax.axis_index("c")
        pltpu.sync_copy(x_hbm.at[pl.ds(wid * RPW, RPW)], idx_v)

        def start(row, s):
            pltpu.async_copy(
                emb_hbm.at[idx_v.at[row, pl.ds(0, GL)]], buf_v.at[s], sems[s]
            )

        def wait(s):
            pltpu.make_async_copy(
                emb_hbm.at[idx_v.at[0, pl.ds(0, GL)]], buf_v.at[s], sems[s]
            ).wait()

        for s in range(NBUF):
            start(s, s)

        def body(i, carry):
            for s in range(NBUF):
                r = i * NBUF + s
                wait(s)
                accs = [buf_v[s, 0, pl.ds(j * 16, 16)] for j in range(NSLICE)]
                for l in range(1, H):
                    for j in range(NSLICE):
                        accs[j] = accs[j] + buf_v[s, l, pl.ds(j * 16, 16)]

                @pl.when(r + NBUF < RPW)
                def _():
                    start(r + NBUF, s)

                for j in range(NSLICE):
                    out_v[r, pl.ds(j * 16, 16)] = accs[j] * (1.0 / H)
            return carry

        lax.fori_loop(0, RPW // NBUF, body, 0)
        pltpu.sync_copy(out_v, out_hbm.at[pl.ds(wid * RPW, RPW)])

    return k(x, emb)


def _linear_tc(pooled, W, b):
    BM = 2048

    def mm(p_ref, w_ref, b_ref, o_ref):
        o_ref[...] = (
            jnp.dot(p_ref[...], w_ref[...], preferred_element_type=jnp.float32)
            + b_ref[...]
        )

    return pl.pallas_call(
        mm,
        grid=(B // BM,),
        in_specs=[
            pl.BlockSpec((BM, D), lambda i: (i, 0)),
            pl.BlockSpec((D, O), lambda i: (0, 0)),
            pl.BlockSpec((1, O), lambda i: (0, 0)),
        ],
        out_specs=pl.BlockSpec((BM, O), lambda i: (i, 0)),
        out_shape=jax.ShapeDtypeStruct((B, O), jnp.float32),
    )(pooled, W, b.reshape(1, O))


def kernel(x, emb, W, b):
    xp = jnp.pad(x.astype(jnp.int32), ((0, 0), (0, HP - H)), mode="wrap")
    pooled = _pooled_sc(xp, emb)
    return _linear_tc(pooled, W, b)
